# baseline (speedup 1.0000x reference)
import functools

import jax
import jax.numpy as jnp
from jax import lax
from jax.experimental import pallas as pl
from jax.experimental.pallas import tpu as pltpu
from jax.experimental.pallas import tpu_sc as plsc

_NC = 2
_NS = 16
_L = 16
_NW = _NC * _NS


@functools.lru_cache(maxsize=None)
def _make_mf(B, D, UP, AP):
    BPW = B // _NW
    C = 128
    NCHUNK = BPW // C
    NG = C // _L

    mesh = plsc.VectorSubcoreMesh(core_axis_name="c", subcore_axis_name="s")

    @functools.partial(
        pl.kernel,
        mesh=mesh,
        compiler_params=pltpu.CompilerParams(needs_layout_passes=False),
        out_type=jax.ShapeDtypeStruct((B,), jnp.float32),
        scratch_types=[
            pltpu.VMEM((BPW,), jnp.int32),
            pltpu.VMEM((BPW,), jnp.int32),
            pltpu.VMEM((2, C, D), jnp.float32),
            pltpu.VMEM((2, C, D), jnp.float32),
            pltpu.VMEM((BPW,), jnp.float32),
            pltpu.VMEM((BPW,), jnp.float32),
            pltpu.VMEM((BPW,), jnp.float32),
            pltpu.VMEM((C // _L, _L * _L), jnp.float32),
            pltpu.SemaphoreType.DMA,
            pltpu.SemaphoreType.DMA,
            pltpu.SemaphoreType.DMA,
        ],
    )
    def mf(uid_hbm, aid_hbm, uemb_hbm, aemb_hbm, ubias_hbm, abias_hbm,
           out_hbm, uid_v, aid_v, u_v, a_v, ub_v, ab_v, out_v,
           tbuf, sem0, sem1, semb):
        wid = lax.axis_index("s") * _NC + lax.axis_index("c")
        base = wid * BPW
        pltpu.sync_copy(uid_hbm.at[pl.ds(base, BPW)], uid_v)
        pltpu.sync_copy(aid_hbm.at[pl.ds(base, BPW)], aid_v)

        lanes = lax.iota(jnp.int32, _L)
        zerog = lanes * 0

        sems = (sem0, sem1)

        def start_rows(chunk):
            b = chunk % 2
            cb = chunk * C
            du = pltpu.async_copy(
                uemb_hbm.at[uid_v.at[pl.ds(cb, C)]], u_v.at[b], sems[b])
            da = pltpu.async_copy(
                aemb_hbm.at[aid_v.at[pl.ds(cb, C)]], a_v.at[b], sems[b])
            return du, da

        row_descs = [None] * NCHUNK
        row_descs[0] = start_rows(0)
        bias_descs = []
        for chunk in range(NCHUNK):
            cb = chunk * C
            bias_descs.append(pltpu.async_copy(
                ubias_hbm.at[uid_v.at[pl.ds(cb, C)]],
                ub_v.at[pl.ds(cb, C)], semb))
            bias_descs.append(pltpu.async_copy(
                abias_hbm.at[aid_v.at[pl.ds(cb, C)]],
                ab_v.at[pl.ds(cb, C)], semb))
        if NCHUNK > 1:
            row_descs[1] = start_rows(1)
        for d in bias_descs:
            d.wait()

        for chunk in range(NCHUNK):
            buf = chunk % 2
            cb = chunk * C
            du, da = row_descs[chunk]
            du.wait()
            da.wait()

            @plsc.parallel_loop(0, NG, 1)
            def group_body(g, cb=cb, buf=buf):
                rowbase = g * _L
                for rr in range(_L):
                    r = rowbase + rr
                    acc = u_v[buf, r, pl.ds(0, _L)] * a_v[buf, r, pl.ds(0, _L)]
                    for k in range(1, D // _L):
                        acc = acc + (u_v[buf, r, pl.ds(k * _L, _L)]
                                     * a_v[buf, r, pl.ds(k * _L, _L)])
                    tbuf[g, pl.ds(rr * _L, _L)] = acc
                dots = plsc.load_gather(tbuf, [zerog + g, lanes * _L])
                for c in range(1, _L):
                    dots = dots + plsc.load_gather(tbuf,
                                                   [zerog + g, lanes * _L + c])
                off = cb + rowbase
                res = dots + ub_v[pl.ds(off, _L)] + ab_v[pl.ds(off, _L)]
                out_v[pl.ds(off, _L)] = res

            if chunk + 2 < NCHUNK:
                row_descs[chunk + 2] = start_rows(chunk + 2)

        pltpu.sync_copy(out_v, out_hbm.at[pl.ds(base, BPW)])

    return mf


def _flat_pad(bias):
    n = bias.shape[0]
    pad = (-n) % 1024
    return jnp.concatenate(
        [bias, jnp.zeros((pad, 1), bias.dtype)], axis=0).reshape(-1)


def kernel(user_id, anime_id, user_embedding, anime_embedding, user_bias,
           anime_bias, global_bias):
    B = user_id.shape[0]
    U, D = user_embedding.shape
    ub_flat = _flat_pad(user_bias)
    ab_flat = _flat_pad(anime_bias + global_bias)
    mf = _make_mf(B, D, ub_flat.shape[0], ab_flat.shape[0])
    return mf(
        user_id.astype(jnp.int32),
        anime_id.astype(jnp.int32),
        user_embedding,
        anime_embedding,
        ub_flat,
        ab_flat,
    )


# speedup vs baseline: 1.0254x; 1.0254x over previous
import functools

import jax
import jax.numpy as jnp
from jax import lax
from jax.experimental import pallas as pl
from jax.experimental.pallas import tpu as pltpu
from jax.experimental.pallas import tpu_sc as plsc

_NC = 2
_NS = 16
_L = 16
_NW = _NC * _NS


@functools.lru_cache(maxsize=None)
def _make_mf(B, D, UP, AP):
    BPW = B // _NW
    C = 128
    NCHUNK = BPW // C
    NG = C // _L

    mesh = plsc.VectorSubcoreMesh(core_axis_name="c", subcore_axis_name="s")

    @functools.partial(
        pl.kernel,
        mesh=mesh,
        compiler_params=pltpu.CompilerParams(needs_layout_passes=False),
        out_type=jax.ShapeDtypeStruct((B,), jnp.float32),
        scratch_types=[
            pltpu.VMEM((BPW,), jnp.int32),
            pltpu.VMEM((BPW,), jnp.int32),
            pltpu.VMEM((2, C, D), jnp.float32),
            pltpu.VMEM((2, C, D), jnp.float32),
            pltpu.VMEM((BPW,), jnp.float32),
            pltpu.VMEM((BPW,), jnp.float32),
            pltpu.VMEM((BPW,), jnp.float32),
            pltpu.VMEM((C // _L, _L * _L), jnp.float32),
            pltpu.SemaphoreType.DMA,
            pltpu.SemaphoreType.DMA,
            pltpu.SemaphoreType.DMA,
        ],
    )
    def mf(uid_hbm, aid_hbm, uemb_hbm, aemb_hbm, ubias_hbm, abias_hbm,
           out_hbm, uid_v, aid_v, u_v, a_v, ub_v, ab_v, out_v,
           tbuf, sem0, sem1, semb):
        wid = lax.axis_index("s") * _NC + lax.axis_index("c")
        base = wid * BPW
        pltpu.sync_copy(uid_hbm.at[pl.ds(base, BPW)], uid_v)
        pltpu.sync_copy(aid_hbm.at[pl.ds(base, BPW)], aid_v)

        lanes = lax.iota(jnp.int32, _L)
        zerog = lanes * 0

        sems = (sem0, sem1)

        def start_rows(chunk):
            b = chunk % 2
            cb = chunk * C
            du = pltpu.async_copy(
                uemb_hbm.at[uid_v.at[pl.ds(cb, C)]], u_v.at[b], sems[b])
            da = pltpu.async_copy(
                aemb_hbm.at[aid_v.at[pl.ds(cb, C)]], a_v.at[b], sems[b])
            return du, da

        row_descs = [None] * NCHUNK
        row_descs[0] = start_rows(0)
        bias_descs = []
        for chunk in range(NCHUNK):
            cb = chunk * C
            bias_descs.append(pltpu.async_copy(
                ubias_hbm.at[uid_v.at[pl.ds(cb, C)]],
                ub_v.at[pl.ds(cb, C)], semb))
            bias_descs.append(pltpu.async_copy(
                abias_hbm.at[aid_v.at[pl.ds(cb, C)]],
                ab_v.at[pl.ds(cb, C)], semb))
        if NCHUNK > 1:
            row_descs[1] = start_rows(1)
        for d in bias_descs:
            d.wait()

        for chunk in range(NCHUNK):
            buf = chunk % 2
            cb = chunk * C
            du, da = row_descs[chunk]
            du.wait()
            da.wait()

            def group_body(g, _, cb=cb, buf=buf):
                rowbase = g * _L
                for rr in range(_L):
                    r = rowbase + rr
                    acc = u_v[buf, r, pl.ds(0, _L)] * a_v[buf, r, pl.ds(0, _L)]
                    for k in range(1, D // _L):
                        acc = acc + (u_v[buf, r, pl.ds(k * _L, _L)]
                                     * a_v[buf, r, pl.ds(k * _L, _L)])
                    tbuf[g, pl.ds(rr * _L, _L)] = acc
                dots = plsc.load_gather(tbuf, [zerog + g, lanes * _L])
                for c in range(1, _L):
                    dots = dots + plsc.load_gather(tbuf,
                                                   [zerog + g, lanes * _L + c])
                off = cb + rowbase
                res = dots + ub_v[pl.ds(off, _L)] + ab_v[pl.ds(off, _L)]
                out_v[pl.ds(off, _L)] = res
                return 0

            lax.fori_loop(0, NG, group_body, 0)

            if chunk + 2 < NCHUNK:
                row_descs[chunk + 2] = start_rows(chunk + 2)

        pltpu.sync_copy(out_v, out_hbm.at[pl.ds(base, BPW)])

    return mf


def _flat_pad(bias):
    n = bias.shape[0]
    pad = (-n) % 1024
    return jnp.pad(bias.T, ((0, 0), (0, pad))).reshape(-1)


def kernel(user_id, anime_id, user_embedding, anime_embedding, user_bias,
           anime_bias, global_bias):
    B = user_id.shape[0]
    U, D = user_embedding.shape
    ub_flat = _flat_pad(user_bias)
    ab_flat = _flat_pad(anime_bias + global_bias)
    mf = _make_mf(B, D, ub_flat.shape[0], ab_flat.shape[0])
    return mf(
        user_id.astype(jnp.int32),
        anime_id.astype(jnp.int32),
        user_embedding,
        anime_embedding,
        ub_flat,
        ab_flat,
    )


# speedup vs baseline: 1.0564x; 1.0301x over previous
import functools

import jax
import jax.numpy as jnp
from jax import lax
from jax.experimental import pallas as pl
from jax.experimental.pallas import tpu as pltpu
from jax.experimental.pallas import tpu_sc as plsc

_NC = 2
_NS = 16
_L = 16
_NW = _NC * _NS


@functools.lru_cache(maxsize=None)
def _make_dots(B, D):
    BPW = B // _NW
    C = 128
    NCHUNK = BPW // C
    NG = C // _L

    mesh = plsc.VectorSubcoreMesh(core_axis_name="c", subcore_axis_name="s")

    @functools.partial(
        pl.kernel,
        mesh=mesh,
        compiler_params=pltpu.CompilerParams(needs_layout_passes=False),
        out_type=jax.ShapeDtypeStruct((B,), jnp.float32),
        scratch_types=[
            pltpu.VMEM((BPW,), jnp.int32),
            pltpu.VMEM((BPW,), jnp.int32),
            pltpu.VMEM((2, C, D), jnp.float32),
            pltpu.VMEM((2, C, D), jnp.float32),
            pltpu.VMEM((BPW,), jnp.float32),
            pltpu.VMEM((C // _L, _L * _L), jnp.float32),
            pltpu.SemaphoreType.DMA,
            pltpu.SemaphoreType.DMA,
        ],
    )
    def dots_k(uid_hbm, aid_hbm, uemb_hbm, aemb_hbm,
               out_hbm, uid_v, aid_v, u_v, a_v, out_v, tbuf, sem0, sem1):
        wid = lax.axis_index("s") * _NC + lax.axis_index("c")
        base = wid * BPW
        pltpu.sync_copy(uid_hbm.at[pl.ds(base, BPW)], uid_v)
        pltpu.sync_copy(aid_hbm.at[pl.ds(base, BPW)], aid_v)

        lanes = lax.iota(jnp.int32, _L)
        zerog = lanes * 0

        sems = (sem0, sem1)

        def start_rows(chunk):
            b = chunk % 2
            cb = chunk * C
            du = pltpu.async_copy(
                uemb_hbm.at[uid_v.at[pl.ds(cb, C)]], u_v.at[b], sems[b])
            da = pltpu.async_copy(
                aemb_hbm.at[aid_v.at[pl.ds(cb, C)]], a_v.at[b], sems[b])
            return du, da

        row_descs = [None] * NCHUNK
        row_descs[0] = start_rows(0)
        if NCHUNK > 1:
            row_descs[1] = start_rows(1)

        for chunk in range(NCHUNK):
            buf = chunk % 2
            cb = chunk * C
            du, da = row_descs[chunk]
            du.wait()
            da.wait()

            def group_body(g, _, cb=cb, buf=buf):
                rowbase = g * _L
                for rr in range(_L):
                    r = rowbase + rr
                    acc = u_v[buf, r, pl.ds(0, _L)] * a_v[buf, r, pl.ds(0, _L)]
                    for k in range(1, D // _L):
                        acc = acc + (u_v[buf, r, pl.ds(k * _L, _L)]
                                     * a_v[buf, r, pl.ds(k * _L, _L)])
                    tbuf[g, pl.ds(rr * _L, _L)] = acc
                dots = plsc.load_gather(tbuf, [zerog + g, lanes * _L])
                for c in range(1, _L):
                    dots = dots + plsc.load_gather(tbuf,
                                                   [zerog + g, lanes * _L + c])
                out_v[pl.ds(cb + rowbase, _L)] = dots
                return 0

            lax.fori_loop(0, NG, group_body, 0)

            if chunk + 2 < NCHUNK:
                row_descs[chunk + 2] = start_rows(chunk + 2)

        pltpu.sync_copy(out_v, out_hbm.at[pl.ds(base, BPW)])

    return dots_k


@functools.lru_cache(maxsize=None)
def _make_bias_add(B, UP, AP):
    BPW = B // _NW
    C = 128
    NCHUNK = BPW // C

    mesh = plsc.VectorSubcoreMesh(core_axis_name="c", subcore_axis_name="s")

    @functools.partial(
        pl.kernel,
        mesh=mesh,
        compiler_params=pltpu.CompilerParams(needs_layout_passes=False),
        out_type=jax.ShapeDtypeStruct((B,), jnp.float32),
        scratch_types=[
            pltpu.VMEM((BPW,), jnp.int32),
            pltpu.VMEM((BPW,), jnp.int32),
            pltpu.VMEM((BPW,), jnp.float32),
            pltpu.VMEM((BPW,), jnp.float32),
            pltpu.VMEM((BPW,), jnp.float32),
            pltpu.VMEM((BPW,), jnp.float32),
            pltpu.SemaphoreType.DMA,
        ],
    )
    def bias_k(uid_hbm, aid_hbm, ubias_hbm, abias_hbm, dots_hbm,
               out_hbm, uid_v, aid_v, dots_v, ub_v, ab_v, out_v, semb):
        wid = lax.axis_index("s") * _NC + lax.axis_index("c")
        base = wid * BPW
        pltpu.sync_copy(uid_hbm.at[pl.ds(base, BPW)], uid_v)
        pltpu.sync_copy(aid_hbm.at[pl.ds(base, BPW)], aid_v)
        descs = []
        for chunk in range(NCHUNK):
            cb = chunk * C
            descs.append(pltpu.async_copy(
                ubias_hbm.at[uid_v.at[pl.ds(cb, C)]],
                ub_v.at[pl.ds(cb, C)], semb))
            descs.append(pltpu.async_copy(
                abias_hbm.at[aid_v.at[pl.ds(cb, C)]],
                ab_v.at[pl.ds(cb, C)], semb))
        pltpu.sync_copy(dots_hbm.at[pl.ds(base, BPW)], dots_v)
        for d in descs:
            d.wait()

        def group_body(g, _):
            off = g * _L
            out_v[pl.ds(off, _L)] = (dots_v[pl.ds(off, _L)]
                                     + ub_v[pl.ds(off, _L)]
                                     + ab_v[pl.ds(off, _L)])
            return 0

        lax.fori_loop(0, BPW // _L, group_body, 0)
        pltpu.sync_copy(out_v, out_hbm.at[pl.ds(base, BPW)])

    return bias_k


def _flat_pad(bias):
    n = bias.shape[0]
    pad = (-n) % 1024
    return jnp.pad(bias.T, ((0, 0), (0, pad))).reshape(-1)


def kernel(user_id, anime_id, user_embedding, anime_embedding, user_bias,
           anime_bias, global_bias):
    B = user_id.shape[0]
    U, D = user_embedding.shape
    uid32 = user_id.astype(jnp.int32)
    aid32 = anime_id.astype(jnp.int32)
    ub_flat = _flat_pad(user_bias)
    ab_flat = _flat_pad(anime_bias + global_bias)
    dots = _make_dots(B, D)(uid32, aid32, user_embedding, anime_embedding)
    return _make_bias_add(B, ub_flat.shape[0], ab_flat.shape[0])(
        uid32, aid32, ub_flat, ab_flat, dots)
